# baseline (speedup 1.0000x reference)
import numpy as np
import jax
import jax.numpy as jnp
from jax.experimental import pallas as pl
from jax.experimental.pallas import tpu as pltpu

_NUM_CLASSES = 4
_CONF_T = 0.05
_MAX_PER_CLASS = 100
_MAX_DET = 100
_SIGMA = 0.05
_NEG = -3.0e38


def _nms_kernel(pred_ref, anch_ref, outf_ref, outc_ref, *, n_real, npad):
    C = _NUM_CLASSES
    T = _MAX_PER_CLASS

    p = pred_ref[0]
    a = anch_ref[...]

    cxa = a[0:1, :]
    cya = a[1:2, :]
    wa = a[2:3, :]
    ha = a[3:4, :]

    x = p[0:1, :] * wa + cxa
    y = p[1:2, :] * ha + cya
    bw = jnp.exp(p[2:3, :]) * wa
    bh = jnp.exp(p[3:4, :]) * ha
    x1 = x - bw / 2.0
    y1 = y - bh / 2.0
    x2 = x + bw / 2.0
    y2 = y + bh / 2.0
    area = (x2 - x1) * (y2 - y1)

    scores0 = jax.nn.sigmoid(p[4:8, :])

    lane = jax.lax.broadcasted_iota(jnp.int32, (C, npad), 1)
    cid = jax.lax.broadcasted_iota(jnp.int32, (C, npad), 0)
    valid_lane = lane < n_real

    max_score = jnp.max(scores0, axis=0, keepdims=True)
    is_mx = scores0 == max_score
    max_cls = jnp.min(jnp.where(is_mx, cid, C), axis=0, keepdims=True)

    active0_b = (max_score >= _CONF_T) & (max_cls == cid) & valid_lane
    active0 = jnp.where(active0_b, 1.0, 0.0).astype(jnp.float32)

    def body(t, carry):
        scores, active, done, sel_s, sel_v, sx1, sy1, sx2, sy2 = carry
        active_b = active > 0.5
        masked = jnp.where(active_b, scores, -1.0)
        m = jnp.max(masked, axis=1, keepdims=True)
        idx = jnp.min(jnp.where(masked == m, lane, npad), axis=1, keepdims=True)
        ok = jnp.logical_and(done < 0.5, m >= _CONF_T)

        oh = lane == idx
        bx1 = jnp.max(jnp.where(oh, x1, _NEG), axis=1, keepdims=True)
        by1 = jnp.max(jnp.where(oh, y1, _NEG), axis=1, keepdims=True)
        bx2 = jnp.max(jnp.where(oh, x2, _NEG), axis=1, keepdims=True)
        by2 = jnp.max(jnp.where(oh, y2, _NEG), axis=1, keepdims=True)

        ix1 = jnp.maximum(bx1, x1)
        iy1 = jnp.maximum(by1, y1)
        ix2 = jnp.minimum(bx2, x2)
        iy2 = jnp.minimum(by2, y2)
        inter = jnp.maximum(ix2 - ix1, 0.0) * jnp.maximum(iy2 - iy1, 0.0)
        a_sel = (bx2 - bx1) * (by2 - by1)
        union = a_sel + area - inter
        iou = jnp.where(union > 0.0, inter / jnp.maximum(union, 1e-12), 0.0)

        rem = jnp.logical_and(active_b, lane != idx)
        decayed = scores * jnp.exp(-(iou * iou) / _SIGMA)
        new_scores = jnp.where(jnp.logical_and(ok, rem), decayed, scores)
        keep_b = jnp.logical_and(rem, new_scores >= _CONF_T)
        new_active = jnp.where(ok, jnp.where(keep_b, 1.0, 0.0), active)
        new_done = jnp.where(ok, jnp.zeros_like(done), jnp.ones_like(done))

        lt = jax.lax.broadcasted_iota(jnp.int32, (C, 128), 1) == t
        okl = jnp.logical_and(lt, ok)
        sel_s = jnp.where(okl, m, sel_s)
        sel_v = jnp.where(lt, jnp.where(ok, 1.0, 0.0), sel_v)
        sx1 = jnp.where(okl, bx1, sx1)
        sy1 = jnp.where(okl, by1, sy1)
        sx2 = jnp.where(okl, bx2, sx2)
        sy2 = jnp.where(okl, by2, sy2)
        return (new_scores, new_active, new_done, sel_s, sel_v, sx1, sy1, sx2, sy2)

    init = (
        scores0,
        active0,
        jnp.zeros((C, 1), dtype=jnp.float32),
        jnp.zeros((C, 128), dtype=jnp.float32),
        jnp.zeros((C, 128), dtype=jnp.float32),
        jnp.zeros((C, 128), dtype=jnp.float32),
        jnp.zeros((C, 128), dtype=jnp.float32),
        jnp.zeros((C, 128), dtype=jnp.float32),
        jnp.zeros((C, 128), dtype=jnp.float32),
    )
    (_, _, _, sel_s, sel_v, sx1, sy1, sx2, sy2) = jax.lax.fori_loop(
        0, T, body, init
    )

    cid8 = jax.lax.broadcasted_iota(jnp.int32, (C, 128), 0)
    g = cid8 * 128 + jax.lax.broadcasted_iota(jnp.int32, (C, 128), 1)
    g_f = g.astype(jnp.float32)
    nvalid = jnp.sum(sel_v)
    case_b = nvalid > float(_MAX_DET)

    primary = jnp.where(case_b, sel_s, -g_f)
    l128 = jax.lax.broadcasted_iota(jnp.int32, (1, 128), 1)

    def mbody(j, carry):
        R, ox1, oy1, ox2, oy2, osc, ocl = carry
        R_b = R > 0.5
        pm = jnp.where(R_b, primary, _NEG)
        m2 = jnp.max(pm)
        any_rem = m2 > (_NEG * 0.5)
        cand = jnp.logical_and(R_b, pm == m2)
        g_sel = jnp.min(jnp.where(cand, g, 1 << 30))
        oh2 = g == g_sel
        vx1 = jnp.max(jnp.where(oh2, sx1, _NEG))
        vy1 = jnp.max(jnp.where(oh2, sy1, _NEG))
        vx2 = jnp.max(jnp.where(oh2, sx2, _NEG))
        vy2 = jnp.max(jnp.where(oh2, sy2, _NEG))
        vsc = jnp.max(jnp.where(oh2, sel_s, _NEG))
        vcl = jnp.max(jnp.where(oh2, cid8, -1))
        new_R = jnp.where(jnp.logical_and(oh2, any_rem), 0.0, R)
        ohj = jnp.logical_and(l128 == j, any_rem)
        ox1 = jnp.where(ohj, vx1, ox1)
        oy1 = jnp.where(ohj, vy1, oy1)
        ox2 = jnp.where(ohj, vx2, ox2)
        oy2 = jnp.where(ohj, vy2, oy2)
        osc = jnp.where(ohj, vsc, osc)
        ocl = jnp.where(ohj, vcl, ocl)
        return (new_R, ox1, oy1, ox2, oy2, osc, ocl)

    minit = (
        sel_v,
        jnp.zeros((1, 128), dtype=jnp.float32),
        jnp.zeros((1, 128), dtype=jnp.float32),
        jnp.zeros((1, 128), dtype=jnp.float32),
        jnp.zeros((1, 128), dtype=jnp.float32),
        jnp.zeros((1, 128), dtype=jnp.float32),
        jnp.full((1, 128), -1, dtype=jnp.int32),
    )
    (_, ox1, oy1, ox2, oy2, osc, ocl) = jax.lax.fori_loop(0, _MAX_DET, mbody, minit)

    zf = jnp.zeros((3, 128), dtype=jnp.float32)
    outf_ref[0] = jnp.concatenate([ox1, oy1, ox2, oy2, osc, zf], axis=0)
    zi = jnp.zeros((7, 128), dtype=jnp.int32)
    outc_ref[0] = jnp.concatenate([ocl, zi], axis=0)


def kernel(predictions, anchor_boxes):
    B, n, _ = predictions.shape
    npad = ((n + 127) // 128) * 128

    predT = jnp.transpose(predictions, (0, 2, 1))
    predT = jnp.pad(predT, ((0, 0), (0, 0), (0, npad - n)))
    anchT = jnp.pad(anchor_boxes.T, ((0, 0), (0, npad - n)))

    import functools
    kfn = functools.partial(_nms_kernel, n_real=n, npad=npad)
    outf, outc = pl.pallas_call(
        kfn,
        grid=(B,),
        in_specs=[
            pl.BlockSpec((1, 8, npad), lambda b: (b, 0, 0)),
            pl.BlockSpec((4, npad), lambda b: (0, 0)),
        ],
        out_specs=[
            pl.BlockSpec((1, 8, 128), lambda b: (b, 0, 0)),
            pl.BlockSpec((1, 8, 128), lambda b: (b, 0, 0)),
        ],
        out_shape=[
            jax.ShapeDtypeStruct((B, 8, 128), jnp.float32),
            jax.ShapeDtypeStruct((B, 8, 128), jnp.int32),
        ],
        compiler_params=pltpu.CompilerParams(
            dimension_semantics=("parallel",),
        ),
    )(predT, anchT)

    M = _MAX_DET
    boxes = jnp.stack(
        [outf[:, 0, :M], outf[:, 1, :M], outf[:, 2, :M], outf[:, 3, :M]], axis=-1
    )
    scores = outf[:, 4, :M]
    classes = outc[:, 0, :M]
    valid = jnp.sum((classes > -1).astype(jnp.int32), axis=1)
    idt = jax.dtypes.canonicalize_dtype(np.int64)
    return (
        valid.astype(jnp.int32),
        boxes.astype(jnp.float32),
        scores.astype(jnp.float32),
        classes.astype(idt),
    )


# speedup vs baseline: 2.5250x; 2.5250x over previous
import functools

import numpy as np
import jax
import jax.numpy as jnp
from jax.experimental import pallas as pl
from jax.experimental.pallas import tpu as pltpu

_NUM_CLASSES = 4
_CONF_T = 0.05
_MAX_PER_CLASS = 100
_MAX_DET = 100
_SIGMA = 0.05
_NEG = -3.0e38
_BIG = 2**30


def _nms_kernel(pred_ref, anch_ref, outf_ref, outc_ref,
                x1_ref, y1_ref, x2_ref, y2_ref, *, n_real, ROWS):
    C = _NUM_CLASSES
    T = _MAX_PER_CLASS

    def ch(ref, k):
        return ref[k * ROWS:(k + 1) * ROWS, :]

    cxa = ch(anch_ref, 0)
    cya = ch(anch_ref, 1)
    wa = ch(anch_ref, 2)
    ha = ch(anch_ref, 3)

    p = pred_ref[0]
    x = ch(p, 0) * wa + cxa
    y = ch(p, 1) * ha + cya
    bw = jnp.exp(ch(p, 2)) * wa
    bh = jnp.exp(ch(p, 3)) * ha
    x1 = x - bw / 2.0
    y1 = y - bh / 2.0
    x2 = x + bw / 2.0
    y2 = y + bh / 2.0
    area = (x2 - x1) * (y2 - y1)

    x1_ref[...] = x1
    y1_ref[...] = y1
    x2_ref[...] = x2
    y2_ref[...] = y2

    s0 = jax.nn.sigmoid(ch(p, 4))
    s1 = jax.nn.sigmoid(ch(p, 5))
    s2 = jax.nn.sigmoid(ch(p, 6))
    s3 = jax.nn.sigmoid(ch(p, 7))
    mx = jnp.maximum(jnp.maximum(s0, s1), jnp.maximum(s2, s3))

    rowi = jax.lax.broadcasted_iota(jnp.int32, (ROWS, 128), 0)
    lanei = jax.lax.broadcasted_iota(jnp.int32, (ROWS, 128), 1)
    flatw = rowi * 128 + lanei
    valid = flatw < n_real

    cls = jnp.where(
        s0 == mx, 0,
        jnp.where(s1 == mx, 1, jnp.where(s2 == mx, 2, 3)),
    ).astype(jnp.int32)
    cmask = [cls == c for c in range(C)]

    active0 = jnp.where((mx >= _CONF_T) & valid, 1.0, 0.0).astype(jnp.float32)

    cid4 = jax.lax.broadcasted_iota(jnp.int32, (C, 1), 0)
    lane128 = jax.lax.broadcasted_iota(jnp.int32, (C, 128), 1)
    l128v = jax.lax.broadcasted_iota(jnp.int32, (1, 128), 1)

    def pick(ref, r, li):
        row = ref[pl.ds(r, 1), :]
        return jnp.max(jnp.where(l128v == li, row, _NEG))

    def sel4(v0, v1, v2, v3):
        return jnp.where(
            cid4 == 0, v0, jnp.where(cid4 == 1, v1, jnp.where(cid4 == 2, v2, v3))
        )

    def chain(m0, m1, m2, v0, v1, v2, v3):
        return jnp.where(m0, v0, jnp.where(m1, v1, jnp.where(m2, v2, v3)))

    def body(t, carry):
        (score, active, d0, d1, d2, d3, sel_s, sel_v, sx1, sy1, sx2, sy2) = carry
        dc = (d0, d1, d2, d3)
        active_b = active > 0.5
        base = jnp.where(active_b, score, -1.0)

        ms, oks, fis, bxs = [], [], [], []
        for c in range(C):
            mc = jnp.max(jnp.where(cmask[c], base, -1.0))
            okc = jnp.logical_and(dc[c] < 0.5, mc >= _CONF_T)
            eq = jnp.logical_and(base == mc, cmask[c])
            fic = jnp.min(jnp.where(eq, flatw, _BIG))
            fic = jnp.where(okc, fic, 0)
            r = fic // 128
            li = fic % 128
            c_x1 = pick(x1_ref, r, li)
            c_y1 = pick(y1_ref, r, li)
            c_x2 = pick(x2_ref, r, li)
            c_y2 = pick(y2_ref, r, li)
            ms.append(mc)
            oks.append(okc)
            fis.append(fic)
            bxs.append((c_x1, c_y1, c_x2, c_y2))

        okf = [jnp.where(o, 1.0, 0.0) for o in oks]
        a_c = [(b[2] - b[0]) * (b[3] - b[1]) for b in bxs]

        m0, m1, m2 = cmask[0], cmask[1], cmask[2]
        okany = chain(m0, m1, m2, okf[0], okf[1], okf[2], okf[3]) > 0.5
        bx1a = chain(m0, m1, m2, bxs[0][0], bxs[1][0], bxs[2][0], bxs[3][0])
        by1a = chain(m0, m1, m2, bxs[0][1], bxs[1][1], bxs[2][1], bxs[3][1])
        bx2a = chain(m0, m1, m2, bxs[0][2], bxs[1][2], bxs[2][2], bxs[3][2])
        by2a = chain(m0, m1, m2, bxs[0][3], bxs[1][3], bxs[2][3], bxs[3][3])
        asel = chain(m0, m1, m2, a_c[0], a_c[1], a_c[2], a_c[3])
        fia = chain(m0, m1, m2, fis[0], fis[1], fis[2], fis[3])

        ix1 = jnp.maximum(bx1a, x1)
        iy1 = jnp.maximum(by1a, y1)
        ix2 = jnp.minimum(bx2a, x2)
        iy2 = jnp.minimum(by2a, y2)
        inter = jnp.maximum(ix2 - ix1, 0.0) * jnp.maximum(iy2 - iy1, 0.0)
        union = asel + area - inter
        iou = jnp.where(union > 0.0, inter / jnp.maximum(union, 1e-12), 0.0)

        rem = jnp.logical_and(active_b, flatw != fia)
        decayed = score * jnp.exp(-(iou * iou) / _SIGMA)
        new_score = jnp.where(jnp.logical_and(okany, rem), decayed, score)
        keep = jnp.logical_and(rem, new_score >= _CONF_T)
        new_active = jnp.where(okany, jnp.where(keep, 1.0, 0.0), active)

        nd = [jnp.where(o, jnp.float32(0.0), jnp.float32(1.0)) for o in oks]

        m_v = sel4(ms[0], ms[1], ms[2], ms[3])
        ok_v = sel4(okf[0], okf[1], okf[2], okf[3]) > 0.5
        x1_v = sel4(bxs[0][0], bxs[1][0], bxs[2][0], bxs[3][0])
        y1_v = sel4(bxs[0][1], bxs[1][1], bxs[2][1], bxs[3][1])
        x2_v = sel4(bxs[0][2], bxs[1][2], bxs[2][2], bxs[3][2])
        y2_v = sel4(bxs[0][3], bxs[1][3], bxs[2][3], bxs[3][3])

        lt = lane128 == t
        okl = jnp.logical_and(lt, ok_v)
        sel_s = jnp.where(okl, m_v, sel_s)
        sel_v = jnp.where(lt, jnp.where(ok_v, 1.0, 0.0), sel_v)
        sx1 = jnp.where(okl, x1_v, sx1)
        sy1 = jnp.where(okl, y1_v, sy1)
        sx2 = jnp.where(okl, x2_v, sx2)
        sy2 = jnp.where(okl, y2_v, sy2)
        return (new_score, new_active, nd[0], nd[1], nd[2], nd[3],
                sel_s, sel_v, sx1, sy1, sx2, sy2)

    z = jnp.float32(0.0)
    init = (
        mx,
        active0,
        z, z, z, z,
        jnp.zeros((C, 128), dtype=jnp.float32),
        jnp.zeros((C, 128), dtype=jnp.float32),
        jnp.zeros((C, 128), dtype=jnp.float32),
        jnp.zeros((C, 128), dtype=jnp.float32),
        jnp.zeros((C, 128), dtype=jnp.float32),
        jnp.zeros((C, 128), dtype=jnp.float32),
    )
    out = jax.lax.fori_loop(0, T, body, init)
    sel_s, sel_v, sx1, sy1, sx2, sy2 = out[6:]

    cid8 = jax.lax.broadcasted_iota(jnp.int32, (C, 128), 0)
    g = cid8 * 128 + jax.lax.broadcasted_iota(jnp.int32, (C, 128), 1)
    g_f = g.astype(jnp.float32)
    nvalid = jnp.sum(sel_v)
    case_b = nvalid > float(_MAX_DET)

    primary = jnp.where(case_b, sel_s, -g_f)
    l128 = jax.lax.broadcasted_iota(jnp.int32, (1, 128), 1)

    def mbody(j, carry):
        R, ox1, oy1, ox2, oy2, osc, ocl = carry
        R_b = R > 0.5
        pm = jnp.where(R_b, primary, _NEG)
        m2 = jnp.max(pm)
        any_rem = m2 > (_NEG * 0.5)
        cand = jnp.logical_and(R_b, pm == m2)
        g_sel = jnp.min(jnp.where(cand, g, 1 << 30))
        oh2 = g == g_sel
        vx1 = jnp.max(jnp.where(oh2, sx1, _NEG))
        vy1 = jnp.max(jnp.where(oh2, sy1, _NEG))
        vx2 = jnp.max(jnp.where(oh2, sx2, _NEG))
        vy2 = jnp.max(jnp.where(oh2, sy2, _NEG))
        vsc = jnp.max(jnp.where(oh2, sel_s, _NEG))
        vcl = jnp.max(jnp.where(oh2, cid8, -1))
        new_R = jnp.where(jnp.logical_and(oh2, any_rem), 0.0, R)
        ohj = jnp.logical_and(l128 == j, any_rem)
        ox1 = jnp.where(ohj, vx1, ox1)
        oy1 = jnp.where(ohj, vy1, oy1)
        ox2 = jnp.where(ohj, vx2, ox2)
        oy2 = jnp.where(ohj, vy2, oy2)
        osc = jnp.where(ohj, vsc, osc)
        ocl = jnp.where(ohj, vcl, ocl)
        return (new_R, ox1, oy1, ox2, oy2, osc, ocl)

    minit = (
        sel_v,
        jnp.zeros((1, 128), dtype=jnp.float32),
        jnp.zeros((1, 128), dtype=jnp.float32),
        jnp.zeros((1, 128), dtype=jnp.float32),
        jnp.zeros((1, 128), dtype=jnp.float32),
        jnp.zeros((1, 128), dtype=jnp.float32),
        jnp.full((1, 128), -1, dtype=jnp.int32),
    )
    (_, ox1, oy1, ox2, oy2, osc, ocl) = jax.lax.fori_loop(0, _MAX_DET, mbody, minit)

    zf = jnp.zeros((3, 128), dtype=jnp.float32)
    outf_ref[0] = jnp.concatenate([ox1, oy1, ox2, oy2, osc, zf], axis=0)
    zi = jnp.zeros((7, 128), dtype=jnp.int32)
    outc_ref[0] = jnp.concatenate([ocl, zi], axis=0)


def kernel(predictions, anchor_boxes):
    B, n, _ = predictions.shape
    npad = ((n + 1023) // 1024) * 1024
    ROWS = npad // 128

    predT = jnp.transpose(predictions, (0, 2, 1))
    predT = jnp.pad(predT, ((0, 0), (0, 0), (0, npad - n)))
    predR = predT.reshape(B, 8, ROWS, 128).reshape(B, 8 * ROWS, 128)
    anchT = jnp.pad(anchor_boxes.T, ((0, 0), (0, npad - n)))
    anchR = anchT.reshape(4, ROWS, 128).reshape(4 * ROWS, 128)

    kfn = functools.partial(_nms_kernel, n_real=n, ROWS=ROWS)
    outf, outc = pl.pallas_call(
        kfn,
        grid=(B,),
        in_specs=[
            pl.BlockSpec((1, 8 * ROWS, 128), lambda b: (b, 0, 0)),
            pl.BlockSpec((4 * ROWS, 128), lambda b: (0, 0)),
        ],
        out_specs=[
            pl.BlockSpec((1, 8, 128), lambda b: (b, 0, 0)),
            pl.BlockSpec((1, 8, 128), lambda b: (b, 0, 0)),
        ],
        out_shape=[
            jax.ShapeDtypeStruct((B, 8, 128), jnp.float32),
            jax.ShapeDtypeStruct((B, 8, 128), jnp.int32),
        ],
        scratch_shapes=[
            pltpu.VMEM((ROWS, 128), jnp.float32),
            pltpu.VMEM((ROWS, 128), jnp.float32),
            pltpu.VMEM((ROWS, 128), jnp.float32),
            pltpu.VMEM((ROWS, 128), jnp.float32),
        ],
        compiler_params=pltpu.CompilerParams(
            dimension_semantics=("arbitrary",),
        ),
    )(predR, anchR)

    M = _MAX_DET
    boxes = jnp.stack(
        [outf[:, 0, :M], outf[:, 1, :M], outf[:, 2, :M], outf[:, 3, :M]], axis=-1
    )
    scores = outf[:, 4, :M]
    classes = outc[:, 0, :M]
    valid = jnp.sum((classes > -1).astype(jnp.int32), axis=1)
    idt = jax.dtypes.canonicalize_dtype(np.int64)
    return (
        valid.astype(jnp.int32),
        boxes.astype(jnp.float32),
        scores.astype(jnp.float32),
        classes.astype(idt),
    )


# speedup vs baseline: 2.5848x; 1.0237x over previous
import functools

import numpy as np
import jax
import jax.numpy as jnp
from jax.experimental import pallas as pl
from jax.experimental.pallas import tpu as pltpu

_NUM_CLASSES = 4
_CONF_T = 0.05
_MAX_PER_CLASS = 100
_MAX_DET = 100
_SIGMA = 0.05
_NEG = -3.0e38
_BIG = 2**30


def _nms_kernel(pred_ref, anch_ref, outf_ref, outc_ref,
                x1_ref, y1_ref, x2_ref, y2_ref, *, n_real, ROWS):
    C = _NUM_CLASSES
    T = _MAX_PER_CLASS

    def ch(ref, k):
        return ref[k * ROWS:(k + 1) * ROWS, :]

    cxa = ch(anch_ref, 0)
    cya = ch(anch_ref, 1)
    wa = ch(anch_ref, 2)
    ha = ch(anch_ref, 3)

    p = pred_ref[0]
    x = ch(p, 0) * wa + cxa
    y = ch(p, 1) * ha + cya
    bw = jnp.exp(ch(p, 2)) * wa
    bh = jnp.exp(ch(p, 3)) * ha
    x1 = x - bw / 2.0
    y1 = y - bh / 2.0
    x2 = x + bw / 2.0
    y2 = y + bh / 2.0
    area = (x2 - x1) * (y2 - y1)

    x1_ref[...] = x1
    y1_ref[...] = y1
    x2_ref[...] = x2
    y2_ref[...] = y2

    s0 = jax.nn.sigmoid(ch(p, 4))
    s1 = jax.nn.sigmoid(ch(p, 5))
    s2 = jax.nn.sigmoid(ch(p, 6))
    s3 = jax.nn.sigmoid(ch(p, 7))
    mx = jnp.maximum(jnp.maximum(s0, s1), jnp.maximum(s2, s3))

    rowi = jax.lax.broadcasted_iota(jnp.int32, (ROWS, 128), 0)
    lanei = jax.lax.broadcasted_iota(jnp.int32, (ROWS, 128), 1)
    flatw = rowi * 128 + lanei
    valid = flatw < n_real

    cls = jnp.where(
        s0 == mx, 0,
        jnp.where(s1 == mx, 1, jnp.where(s2 == mx, 2, 3)),
    ).astype(jnp.int32)

    score0 = jnp.where((mx >= _CONF_T) & valid, mx, -1.0)

    cid4 = jax.lax.broadcasted_iota(jnp.int32, (C, 1), 0)
    lane128 = jax.lax.broadcasted_iota(jnp.int32, (C, 128), 1)
    l128v = jax.lax.broadcasted_iota(jnp.int32, (1, 128), 1)

    def pick(ref, r, li):
        row = ref[pl.ds(r, 1), :]
        return jnp.max(jnp.where(l128v == li, row, _NEG))

    def sel4(v0, v1, v2, v3):
        return jnp.where(
            cid4 == 0, v0, jnp.where(cid4 == 1, v1, jnp.where(cid4 == 2, v2, v3))
        )

    def chain(m0, m1, m2, v0, v1, v2, v3):
        return jnp.where(m0, v0, jnp.where(m1, v1, jnp.where(m2, v2, v3)))

    def body(t, carry):
        (score, d0, d1, d2, d3, sel_s, sel_v, sx1, sy1, sx2, sy2) = carry
        dc = (d0, d1, d2, d3)
        base = score
        cmask = [cls == c for c in range(C)]
        flat = jax.lax.broadcasted_iota(jnp.int32, (ROWS, 128), 0) * 128 + \
            jax.lax.broadcasted_iota(jnp.int32, (ROWS, 128), 1)

        ms, oks, fis, bxs = [], [], [], []
        for c in range(C):
            mc = jnp.max(jnp.where(cmask[c], base, -1.0))
            okc = jnp.logical_and(dc[c] < 0.5, mc >= _CONF_T)
            eq = jnp.logical_and(base == mc, cmask[c])
            fic = jnp.min(jnp.where(eq, flat, _BIG))
            fic = jnp.where(okc, fic, 0)
            r = fic // 128
            li = fic % 128
            c_x1 = pick(x1_ref, r, li)
            c_y1 = pick(y1_ref, r, li)
            c_x2 = pick(x2_ref, r, li)
            c_y2 = pick(y2_ref, r, li)
            ms.append(mc)
            oks.append(okc)
            fis.append(fic)
            bxs.append((c_x1, c_y1, c_x2, c_y2))

        okf = [jnp.where(o, 1.0, 0.0) for o in oks]
        a_c = [(b[2] - b[0]) * (b[3] - b[1]) for b in bxs]

        m0, m1, m2 = cmask[0], cmask[1], cmask[2]
        okany = chain(m0, m1, m2, okf[0], okf[1], okf[2], okf[3]) > 0.5
        bx1a = chain(m0, m1, m2, bxs[0][0], bxs[1][0], bxs[2][0], bxs[3][0])
        by1a = chain(m0, m1, m2, bxs[0][1], bxs[1][1], bxs[2][1], bxs[3][1])
        bx2a = chain(m0, m1, m2, bxs[0][2], bxs[1][2], bxs[2][2], bxs[3][2])
        by2a = chain(m0, m1, m2, bxs[0][3], bxs[1][3], bxs[2][3], bxs[3][3])
        asel = chain(m0, m1, m2, a_c[0], a_c[1], a_c[2], a_c[3])
        fia = chain(m0, m1, m2, fis[0], fis[1], fis[2], fis[3])

        ix1 = jnp.maximum(bx1a, x1)
        iy1 = jnp.maximum(by1a, y1)
        ix2 = jnp.minimum(bx2a, x2)
        iy2 = jnp.minimum(by2a, y2)
        inter = jnp.maximum(ix2 - ix1, 0.0) * jnp.maximum(iy2 - iy1, 0.0)
        union = asel + ((x2 - x1) * (y2 - y1)) - inter
        iou = jnp.where(union > 0.0, inter / jnp.maximum(union, 1e-12), 0.0)

        decayed = score * jnp.exp(-(iou * iou) / _SIGMA)
        keep = jnp.logical_and(flat != fia, decayed >= _CONF_T)
        new_score = jnp.where(
            okany, jnp.where(keep, decayed, -1.0), score)

        nd = [jnp.where(o, jnp.float32(0.0), jnp.float32(1.0)) for o in oks]

        m_v = sel4(ms[0], ms[1], ms[2], ms[3])
        ok_v = sel4(okf[0], okf[1], okf[2], okf[3]) > 0.5
        x1_v = sel4(bxs[0][0], bxs[1][0], bxs[2][0], bxs[3][0])
        y1_v = sel4(bxs[0][1], bxs[1][1], bxs[2][1], bxs[3][1])
        x2_v = sel4(bxs[0][2], bxs[1][2], bxs[2][2], bxs[3][2])
        y2_v = sel4(bxs[0][3], bxs[1][3], bxs[2][3], bxs[3][3])

        lt = lane128 == t
        okl = jnp.logical_and(lt, ok_v)
        sel_s = jnp.where(okl, m_v, sel_s)
        sel_v = jnp.where(lt, jnp.where(ok_v, 1.0, 0.0), sel_v)
        sx1 = jnp.where(okl, x1_v, sx1)
        sy1 = jnp.where(okl, y1_v, sy1)
        sx2 = jnp.where(okl, x2_v, sx2)
        sy2 = jnp.where(okl, y2_v, sy2)
        return (new_score, nd[0], nd[1], nd[2], nd[3],
                sel_s, sel_v, sx1, sy1, sx2, sy2)

    z = jnp.float32(0.0)
    init = (
        score0,
        z, z, z, z,
        jnp.zeros((C, 128), dtype=jnp.float32),
        jnp.zeros((C, 128), dtype=jnp.float32),
        jnp.zeros((C, 128), dtype=jnp.float32),
        jnp.zeros((C, 128), dtype=jnp.float32),
        jnp.zeros((C, 128), dtype=jnp.float32),
        jnp.zeros((C, 128), dtype=jnp.float32),
    )
    out = jax.lax.fori_loop(0, T, body, init)
    sel_s, sel_v, sx1, sy1, sx2, sy2 = out[5:]

    cid8 = jax.lax.broadcasted_iota(jnp.int32, (C, 128), 0)
    g = cid8 * 128 + jax.lax.broadcasted_iota(jnp.int32, (C, 128), 1)
    g_f = g.astype(jnp.float32)
    nvalid = jnp.sum(sel_v)
    case_b = nvalid > float(_MAX_DET)

    primary = jnp.where(case_b, sel_s, -g_f)
    l128 = jax.lax.broadcasted_iota(jnp.int32, (1, 128), 1)

    def mbody(j, carry):
        R, ox1, oy1, ox2, oy2, osc, ocl = carry
        R_b = R > 0.5
        pm = jnp.where(R_b, primary, _NEG)
        m2 = jnp.max(pm)
        any_rem = m2 > (_NEG * 0.5)
        cand = jnp.logical_and(R_b, pm == m2)
        g_sel = jnp.min(jnp.where(cand, g, 1 << 30))
        oh2 = g == g_sel
        vx1 = jnp.max(jnp.where(oh2, sx1, _NEG))
        vy1 = jnp.max(jnp.where(oh2, sy1, _NEG))
        vx2 = jnp.max(jnp.where(oh2, sx2, _NEG))
        vy2 = jnp.max(jnp.where(oh2, sy2, _NEG))
        vsc = jnp.max(jnp.where(oh2, sel_s, _NEG))
        vcl = jnp.max(jnp.where(oh2, cid8, -1))
        new_R = jnp.where(jnp.logical_and(oh2, any_rem), 0.0, R)
        ohj = jnp.logical_and(l128 == j, any_rem)
        ox1 = jnp.where(ohj, vx1, ox1)
        oy1 = jnp.where(ohj, vy1, oy1)
        ox2 = jnp.where(ohj, vx2, ox2)
        oy2 = jnp.where(ohj, vy2, oy2)
        osc = jnp.where(ohj, vsc, osc)
        ocl = jnp.where(ohj, vcl, ocl)
        return (new_R, ox1, oy1, ox2, oy2, osc, ocl)

    minit = (
        sel_v,
        jnp.zeros((1, 128), dtype=jnp.float32),
        jnp.zeros((1, 128), dtype=jnp.float32),
        jnp.zeros((1, 128), dtype=jnp.float32),
        jnp.zeros((1, 128), dtype=jnp.float32),
        jnp.zeros((1, 128), dtype=jnp.float32),
        jnp.full((1, 128), -1, dtype=jnp.int32),
    )
    (_, ox1, oy1, ox2, oy2, osc, ocl) = jax.lax.fori_loop(0, _MAX_DET, mbody, minit)

    zf = jnp.zeros((3, 128), dtype=jnp.float32)
    outf_ref[0] = jnp.concatenate([ox1, oy1, ox2, oy2, osc, zf], axis=0)
    zi = jnp.zeros((7, 128), dtype=jnp.int32)
    outc_ref[0] = jnp.concatenate([ocl, zi], axis=0)


def kernel(predictions, anchor_boxes):
    B, n, _ = predictions.shape
    npad = ((n + 1023) // 1024) * 1024
    ROWS = npad // 128

    predT = jnp.transpose(predictions, (0, 2, 1))
    predT = jnp.pad(predT, ((0, 0), (0, 0), (0, npad - n)))
    predR = predT.reshape(B, 8, ROWS, 128).reshape(B, 8 * ROWS, 128)
    anchT = jnp.pad(anchor_boxes.T, ((0, 0), (0, npad - n)))
    anchR = anchT.reshape(4, ROWS, 128).reshape(4 * ROWS, 128)

    kfn = functools.partial(_nms_kernel, n_real=n, ROWS=ROWS)
    outf, outc = pl.pallas_call(
        kfn,
        grid=(B,),
        in_specs=[
            pl.BlockSpec((1, 8 * ROWS, 128), lambda b: (b, 0, 0)),
            pl.BlockSpec((4 * ROWS, 128), lambda b: (0, 0)),
        ],
        out_specs=[
            pl.BlockSpec((1, 8, 128), lambda b: (b, 0, 0)),
            pl.BlockSpec((1, 8, 128), lambda b: (b, 0, 0)),
        ],
        out_shape=[
            jax.ShapeDtypeStruct((B, 8, 128), jnp.float32),
            jax.ShapeDtypeStruct((B, 8, 128), jnp.int32),
        ],
        scratch_shapes=[
            pltpu.VMEM((ROWS, 128), jnp.float32),
            pltpu.VMEM((ROWS, 128), jnp.float32),
            pltpu.VMEM((ROWS, 128), jnp.float32),
            pltpu.VMEM((ROWS, 128), jnp.float32),
        ],
        compiler_params=pltpu.CompilerParams(
            dimension_semantics=("arbitrary",),
        ),
    )(predR, anchR)

    M = _MAX_DET
    boxes = jnp.stack(
        [outf[:, 0, :M], outf[:, 1, :M], outf[:, 2, :M], outf[:, 3, :M]], axis=-1
    )
    scores = outf[:, 4, :M]
    classes = outc[:, 0, :M]
    valid = jnp.sum((classes > -1).astype(jnp.int32), axis=1)
    idt = jax.dtypes.canonicalize_dtype(np.int64)
    return (
        valid.astype(jnp.int32),
        boxes.astype(jnp.float32),
        scores.astype(jnp.float32),
        classes.astype(idt),
    )


# speedup vs baseline: 2.6697x; 1.0328x over previous
import functools

import numpy as np
import jax
import jax.numpy as jnp
from jax.experimental import pallas as pl
from jax.experimental.pallas import tpu as pltpu

_NUM_CLASSES = 4
_CONF_T = 0.05
_MAX_PER_CLASS = 100
_MAX_DET = 100
_SIGMA = 0.05
_NEG = -3.0e38
_BIG = 2**30


def _nms_kernel(pred_ref, anch_ref, outf_ref, outc_ref,
                x1_ref, y1_ref, x2_ref, y2_ref, *, B, n_real, ROWS):
    C = _NUM_CLASSES
    T = _MAX_PER_CLASS
    BR = B * ROWS

    def ch(ref, k):
        return ref[k * BR:(k + 1) * BR, :]

    cxa = ch(anch_ref, 0)
    cya = ch(anch_ref, 1)
    wa = ch(anch_ref, 2)
    ha = ch(anch_ref, 3)

    x = ch(pred_ref, 0) * wa + cxa
    y = ch(pred_ref, 1) * ha + cya
    bw = jnp.exp(ch(pred_ref, 2)) * wa
    bh = jnp.exp(ch(pred_ref, 3)) * ha
    x1 = x - bw / 2.0
    y1 = y - bh / 2.0
    x2 = x + bw / 2.0
    y2 = y + bh / 2.0
    area = (x2 - x1) * (y2 - y1)

    x1_ref[...] = x1
    y1_ref[...] = y1
    x2_ref[...] = x2
    y2_ref[...] = y2

    s0 = jax.nn.sigmoid(ch(pred_ref, 4))
    s1 = jax.nn.sigmoid(ch(pred_ref, 5))
    s2 = jax.nn.sigmoid(ch(pred_ref, 6))
    s3 = jax.nn.sigmoid(ch(pred_ref, 7))
    mx = jnp.maximum(jnp.maximum(s0, s1), jnp.maximum(s2, s3))

    rowi = jax.lax.broadcasted_iota(jnp.int32, (BR, 128), 0)
    lanei = jax.lax.broadcasted_iota(jnp.int32, (BR, 128), 1)
    rloc = rowi - (rowi // ROWS) * ROWS
    flatw = rloc * 128 + lanei
    valid = flatw < n_real

    cls = jnp.where(
        s0 == mx, 0,
        jnp.where(s1 == mx, 1, jnp.where(s2 == mx, 2, 3)),
    ).astype(jnp.int32)

    score0 = jnp.where((mx >= _CONF_T) & valid, mx, -1.0)

    cmask = [cls == c for c in range(C)]
    l128v = jax.lax.broadcasted_iota(jnp.int32, (1, 128), 1)

    def pick(ref, r, li):
        row = ref[pl.ds(r, 1), :]
        return jnp.max(jnp.where(l128v == li, row, _NEG))

    rows32 = jax.lax.broadcasted_iota(jnp.int32, (B * C, 1), 0)
    lane128 = jax.lax.broadcasted_iota(jnp.int32, (B * C, 128), 1)

    def sel_rows(vals):
        out = vals[B * C - 1]
        for i in range(B * C - 2, -1, -1):
            out = jnp.where(rows32 == i, vals[i], out)
        return out

    def bsl(arr, b):
        return arr[b * ROWS:(b + 1) * ROWS, :]

    def body(t, carry):
        (score, done, sel_s, sel_v, sx1, sy1, sx2, sy2) = carry

        ms, oks, fis, bxs = [], [], [], []
        for b in range(B):
            sb = bsl(score, b)
            fb = bsl(flatw, b)
            for c in range(C):
                i = b * C + c
                cm = bsl(cmask[c], b)
                mc = jnp.max(jnp.where(cm, sb, -1.0))
                okc = jnp.logical_and(done[i] < 0.5, mc >= _CONF_T)
                eq = jnp.logical_and(sb == mc, cm)
                fic = jnp.min(jnp.where(eq, fb, _BIG))
                fic = jnp.where(okc, fic, 0)
                r = b * ROWS + fic // 128
                li = fic % 128
                ms.append(mc)
                oks.append(okc)
                fis.append(fic)
                bxs.append((pick(x1_ref, r, li), pick(y1_ref, r, li),
                            pick(x2_ref, r, li), pick(y2_ref, r, li)))

        okf = [jnp.where(o, 1.0, 0.0) for o in oks]
        a_c = [(bx[2] - bx[0]) * (bx[3] - bx[1]) for bx in bxs]

        def chain_b(vals, b):
            v = [vals[b * C + c] for c in range(C)]
            return jnp.where(
                bsl(cmask[0], b), v[0],
                jnp.where(bsl(cmask[1], b), v[1],
                          jnp.where(bsl(cmask[2], b), v[2], v[3])))

        def chain(vals):
            return jnp.concatenate([chain_b(vals, b) for b in range(B)], axis=0)

        okany = chain(okf) > 0.5
        bx1a = chain([bx[0] for bx in bxs])
        by1a = chain([bx[1] for bx in bxs])
        bx2a = chain([bx[2] for bx in bxs])
        by2a = chain([bx[3] for bx in bxs])
        asel = chain(a_c)
        fia = chain(fis)

        ix1 = jnp.maximum(bx1a, x1)
        iy1 = jnp.maximum(by1a, y1)
        ix2 = jnp.minimum(bx2a, x2)
        iy2 = jnp.minimum(by2a, y2)
        inter = jnp.maximum(ix2 - ix1, 0.0) * jnp.maximum(iy2 - iy1, 0.0)
        union = asel + area - inter
        iou = jnp.where(union > 0.0, inter / jnp.maximum(union, 1e-12), 0.0)

        decayed = score * jnp.exp(-(iou * iou) / _SIGMA)
        keep = jnp.logical_and(flatw != fia, decayed >= _CONF_T)
        new_score = jnp.where(okany, jnp.where(keep, decayed, -1.0), score)

        nd = [jnp.where(o, jnp.float32(0.0), jnp.float32(1.0)) for o in oks]

        m_v = sel_rows(ms)
        ok_v = sel_rows(okf) > 0.5
        x1_v = sel_rows([bx[0] for bx in bxs])
        y1_v = sel_rows([bx[1] for bx in bxs])
        x2_v = sel_rows([bx[2] for bx in bxs])
        y2_v = sel_rows([bx[3] for bx in bxs])

        lt = lane128 == t
        okl = jnp.logical_and(lt, ok_v)
        sel_s = jnp.where(okl, m_v, sel_s)
        sel_v = jnp.where(lt, jnp.where(ok_v, 1.0, 0.0), sel_v)
        sx1 = jnp.where(okl, x1_v, sx1)
        sy1 = jnp.where(okl, y1_v, sy1)
        sx2 = jnp.where(okl, x2_v, sx2)
        sy2 = jnp.where(okl, y2_v, sy2)
        return (new_score, nd, sel_s, sel_v, sx1, sy1, sx2, sy2)

    z = jnp.float32(0.0)
    init = (
        score0,
        [z] * (B * C),
        jnp.zeros((B * C, 128), dtype=jnp.float32),
        jnp.zeros((B * C, 128), dtype=jnp.float32),
        jnp.zeros((B * C, 128), dtype=jnp.float32),
        jnp.zeros((B * C, 128), dtype=jnp.float32),
        jnp.zeros((B * C, 128), dtype=jnp.float32),
        jnp.zeros((B * C, 128), dtype=jnp.float32),
    )
    out = jax.lax.fori_loop(0, T, body, init)
    sel_s, sel_v, sx1, sy1, sx2, sy2 = out[2:]

    cid4 = jax.lax.broadcasted_iota(jnp.int32, (C, 128), 0)
    g = cid4 * 128 + jax.lax.broadcasted_iota(jnp.int32, (C, 128), 1)
    g_f = g.astype(jnp.float32)
    l128 = jax.lax.broadcasted_iota(jnp.int32, (1, 128), 1)

    for b in range(B):
        bs = slice(b * C, (b + 1) * C)
        sel_s_b = sel_s[bs, :]
        sel_v_b = sel_v[bs, :]
        sx1_b = sx1[bs, :]
        sy1_b = sy1[bs, :]
        sx2_b = sx2[bs, :]
        sy2_b = sy2[bs, :]
        nvalid = jnp.sum(sel_v_b)
        case_b = nvalid > float(_MAX_DET)
        primary = jnp.where(case_b, sel_s_b, -g_f)

        def mbody(j, carry, primary=primary, sel_s_b=sel_s_b, sx1_b=sx1_b,
                  sy1_b=sy1_b, sx2_b=sx2_b, sy2_b=sy2_b):
            R, ox1, oy1, ox2, oy2, osc, ocl = carry
            R_b = R > 0.5
            pm = jnp.where(R_b, primary, _NEG)
            m2 = jnp.max(pm)
            any_rem = m2 > (_NEG * 0.5)
            cand = jnp.logical_and(R_b, pm == m2)
            g_sel = jnp.min(jnp.where(cand, g, 1 << 30))
            oh2 = g == g_sel
            vx1 = jnp.max(jnp.where(oh2, sx1_b, _NEG))
            vy1 = jnp.max(jnp.where(oh2, sy1_b, _NEG))
            vx2 = jnp.max(jnp.where(oh2, sx2_b, _NEG))
            vy2 = jnp.max(jnp.where(oh2, sy2_b, _NEG))
            vsc = jnp.max(jnp.where(oh2, sel_s_b, _NEG))
            vcl = jnp.max(jnp.where(oh2, cid4, -1))
            new_R = jnp.where(jnp.logical_and(oh2, any_rem), 0.0, R)
            ohj = jnp.logical_and(l128 == j, any_rem)
            ox1 = jnp.where(ohj, vx1, ox1)
            oy1 = jnp.where(ohj, vy1, oy1)
            ox2 = jnp.where(ohj, vx2, ox2)
            oy2 = jnp.where(ohj, vy2, oy2)
            osc = jnp.where(ohj, vsc, osc)
            ocl = jnp.where(ohj, vcl, ocl)
            return (new_R, ox1, oy1, ox2, oy2, osc, ocl)

        minit = (
            sel_v_b,
            jnp.zeros((1, 128), dtype=jnp.float32),
            jnp.zeros((1, 128), dtype=jnp.float32),
            jnp.zeros((1, 128), dtype=jnp.float32),
            jnp.zeros((1, 128), dtype=jnp.float32),
            jnp.zeros((1, 128), dtype=jnp.float32),
            jnp.full((1, 128), -1, dtype=jnp.int32),
        )
        (_, ox1, oy1, ox2, oy2, osc, ocl) = jax.lax.fori_loop(
            0, _MAX_DET, mbody, minit)

        zf = jnp.zeros((3, 128), dtype=jnp.float32)
        outf_ref[b] = jnp.concatenate([ox1, oy1, ox2, oy2, osc, zf], axis=0)
        zi = jnp.zeros((7, 128), dtype=jnp.int32)
        outc_ref[b] = jnp.concatenate([ocl, zi], axis=0)


def kernel(predictions, anchor_boxes):
    B, n, _ = predictions.shape
    npad = ((n + 1023) // 1024) * 1024
    ROWS = npad // 128
    BR = B * ROWS

    predT = jnp.transpose(predictions, (2, 0, 1))
    predT = jnp.pad(predT, ((0, 0), (0, 0), (0, npad - n)))
    predR = predT.reshape(8, BR, 128).reshape(8 * BR, 128)
    anchT = jnp.pad(anchor_boxes.T, ((0, 0), (0, npad - n)))
    anchR = jnp.tile(anchT[:, None, :], (1, B, 1)).reshape(4 * BR, 128)

    kfn = functools.partial(_nms_kernel, B=B, n_real=n, ROWS=ROWS)
    outf, outc = pl.pallas_call(
        kfn,
        grid=(1,),
        in_specs=[
            pl.BlockSpec((8 * BR, 128), lambda i: (0, 0)),
            pl.BlockSpec((4 * BR, 128), lambda i: (0, 0)),
        ],
        out_specs=[
            pl.BlockSpec((B, 8, 128), lambda i: (0, 0, 0)),
            pl.BlockSpec((B, 8, 128), lambda i: (0, 0, 0)),
        ],
        out_shape=[
            jax.ShapeDtypeStruct((B, 8, 128), jnp.float32),
            jax.ShapeDtypeStruct((B, 8, 128), jnp.int32),
        ],
        scratch_shapes=[
            pltpu.VMEM((BR, 128), jnp.float32),
            pltpu.VMEM((BR, 128), jnp.float32),
            pltpu.VMEM((BR, 128), jnp.float32),
            pltpu.VMEM((BR, 128), jnp.float32),
        ],
        compiler_params=pltpu.CompilerParams(
            dimension_semantics=("arbitrary",),
        ),
    )(predR, anchR)

    M = _MAX_DET
    boxes = jnp.stack(
        [outf[:, 0, :M], outf[:, 1, :M], outf[:, 2, :M], outf[:, 3, :M]], axis=-1
    )
    scores = outf[:, 4, :M]
    classes = outc[:, 0, :M]
    valid = jnp.sum((classes > -1).astype(jnp.int32), axis=1)
    idt = jax.dtypes.canonicalize_dtype(np.int64)
    return (
        valid.astype(jnp.int32),
        boxes.astype(jnp.float32),
        scores.astype(jnp.float32),
        classes.astype(idt),
    )
